# baseline (speedup 1.0000x reference)
import jax
import jax.numpy as jnp
from jax.experimental import pallas as pl
from jax.experimental.pallas import tpu as pltpu

_TILE = 1024
_KSPLIT = 2


def _fused(x_ref, wk_ref, w_ref, pb_ref, out_ref, y_ref, carry_ref):
    i = pl.program_id(0)
    k = pl.program_id(1)
    t = x_ref.shape[0]
    m = pb_ref.shape[1]

    @pl.when((i == 0) & (k == 0))
    def _():
        carry_ref[...] = jnp.zeros_like(carry_ref)

    part = jax.lax.dot_general(x_ref[...], wk_ref[...], (((1,), (1,)), ((), ())),
                               preferred_element_type=jnp.float32)

    @pl.when(k == 0)
    def _():
        y_ref[...] = part

    @pl.when(k == _KSPLIT - 1)
    def _():
        y = y_ref[...] + part
        pb = pb_ref[...]
        prev = carry_ref[...]
        full01 = jnp.concatenate([prev, y[:, 0:2 * m]], axis=0)
        carry_ref[...] = y[t - 2:t, 0:2 * m]

        s0 = full01[0:t, 0:m] + pb[0][None, :]
        s1 = full01[1:t + 1, m:2 * m] + pb[1][None, :]
        s2 = y[:, 2 * m:3 * m] + pb[2][None, :]

        o0 = (s0 >= s1) & (s0 >= s2)
        o1 = jnp.logical_not(o0) & (s1 >= s2)
        f = jnp.concatenate([o0.astype(jnp.bfloat16),
                             o1.astype(jnp.bfloat16)], axis=1)

        w = w_ref[...]
        mem2 = w[2 * m:3 * m]
        dcat = (w[0:2 * m] - jnp.concatenate([mem2, mem2], axis=0)
                ).astype(jnp.bfloat16)
        base = jnp.sum(mem2, axis=0)[None, :]

        out = jax.lax.dot_general(f, dcat, (((1,), (0,)), ((), ())),
                                  preferred_element_type=jnp.float32)
        out_ref[...] = out + base


def kernel(x, memory, pos_bias):
    b, l, d = x.shape
    m, n = pos_bias.shape
    dk = d // _KSPLIT
    w = memory.transpose(1, 0, 2).reshape(n * m, d)
    pb_t = pos_bias.T
    out = pl.pallas_call(
        _fused,
        grid=(l // _TILE, _KSPLIT),
        in_specs=[
            pl.BlockSpec((_TILE, dk), lambda i, k: (i, k)),
            pl.BlockSpec((n * m, dk), lambda i, k: (0, k)),
            pl.BlockSpec((n * m, d), lambda i, k: (0, 0)),
            pl.BlockSpec((n, m), lambda i, k: (0, 0)),
        ],
        out_specs=pl.BlockSpec((_TILE, d), lambda i, k: (i, 0)),
        scratch_shapes=[pltpu.VMEM((_TILE, n * m), jnp.float32),
                        pltpu.VMEM((2, 2 * m), jnp.float32)],
        out_shape=jax.ShapeDtypeStruct((l, d), jnp.float32),
    )(x[0], w, w, pb_t)
    return out[None]


# speedup vs baseline: 1.0163x; 1.0163x over previous
import jax
import jax.numpy as jnp
from jax.experimental import pallas as pl
from jax.experimental.pallas import tpu as pltpu

_TILE = 1024
_OSPLIT = 2


def _fused(x_ref, w_ref, pb_ref, out_ref, f_ref, carry_ref):
    i = pl.program_id(0)
    k = pl.program_id(1)
    t = x_ref.shape[0]
    m = pb_ref.shape[1]
    w = w_ref[...]

    @pl.when((i == 0) & (k == 0))
    def _():
        carry_ref[...] = jnp.zeros_like(carry_ref)

    @pl.when(k == 0)
    def _():
        y = jax.lax.dot_general(x_ref[...], w, (((1,), (1,)), ((), ())),
                                preferred_element_type=jnp.float32)
        pb = pb_ref[...]
        prev = carry_ref[...]
        full01 = jnp.concatenate([prev, y[:, 0:2 * m]], axis=0)
        carry_ref[...] = y[t - 2:t, 0:2 * m]

        s0 = full01[0:t, 0:m] + pb[0][None, :]
        s1 = full01[1:t + 1, m:2 * m] + pb[1][None, :]
        s2 = y[:, 2 * m:3 * m] + pb[2][None, :]

        o0 = (s0 >= s1) & (s0 >= s2)
        o1 = jnp.logical_not(o0) & (s1 >= s2)
        f_ref[...] = jnp.concatenate([o0.astype(jnp.bfloat16),
                                      o1.astype(jnp.bfloat16)], axis=1)

    dk = out_ref.shape[1]
    mem2 = w_ref[2 * m:3 * m, pl.ds(k * dk, dk)]
    dcat = (w_ref[0:2 * m, pl.ds(k * dk, dk)]
            - jnp.concatenate([mem2, mem2], axis=0)
            ).astype(jnp.bfloat16)
    base = jnp.sum(mem2, axis=0)[None, :]

    out = jax.lax.dot_general(f_ref[...], dcat, (((1,), (0,)), ((), ())),
                              preferred_element_type=jnp.float32)
    out_ref[...] = out + base


def kernel(x, memory, pos_bias):
    b, l, d = x.shape
    m, n = pos_bias.shape
    dk = d // _OSPLIT
    w = memory.transpose(1, 0, 2).reshape(n * m, d)
    pb_t = pos_bias.T
    out = pl.pallas_call(
        _fused,
        grid=(l // _TILE, _OSPLIT),
        in_specs=[
            pl.BlockSpec((_TILE, d), lambda i, k: (i, 0)),
            pl.BlockSpec((n * m, d), lambda i, k: (0, 0)),
            pl.BlockSpec((n, m), lambda i, k: (0, 0)),
        ],
        out_specs=pl.BlockSpec((_TILE, dk), lambda i, k: (i, k)),
        scratch_shapes=[pltpu.VMEM((_TILE, 2 * m), jnp.bfloat16),
                        pltpu.VMEM((2, 2 * m), jnp.float32)],
        out_shape=jax.ShapeDtypeStruct((l, d), jnp.float32),
    )(x[0], w, pb_t)
    return out[None]


# speedup vs baseline: 1.2052x; 1.1859x over previous
import jax
import jax.numpy as jnp
from jax.experimental import pallas as pl
from jax.experimental.pallas import tpu as pltpu

_TILE = 1024


def _fused(x_ref, w_ref, pb_ref, out_ref, carry_ref):
    i = pl.program_id(0)
    t = x_ref.shape[0]
    m = pb_ref.shape[1]

    @pl.when(i == 0)
    def _():
        carry_ref[...] = jnp.zeros_like(carry_ref)

    w = w_ref[...]
    y = jax.lax.dot_general(x_ref[...], w, (((1,), (1,)), ((), ())),
                            preferred_element_type=jnp.float32)
    pb = pb_ref[...]
    prev = carry_ref[...]
    full01 = jnp.concatenate([prev, y[:, 0:2 * m]], axis=0)
    carry_ref[...] = y[t - 2:t, 0:2 * m]

    s0 = full01[0:t, 0:m] + pb[0][None, :]
    s1 = full01[1:t + 1, m:2 * m] + pb[1][None, :]
    s2 = y[:, 2 * m:3 * m] + pb[2][None, :]

    o0 = (s0 >= s1) & (s0 >= s2)
    n0 = jnp.logical_not(o0)
    o1 = n0 & (s1 >= s2)
    o2 = n0 & jnp.logical_not(s1 >= s2)
    f = jnp.concatenate([o0.astype(jnp.bfloat16),
                         o1.astype(jnp.bfloat16),
                         o2.astype(jnp.bfloat16)], axis=1)

    out_ref[...] = jax.lax.dot_general(
        f, w.astype(jnp.bfloat16), (((1,), (0,)), ((), ())),
        preferred_element_type=jnp.float32)


def kernel(x, memory, pos_bias):
    b, l, d = x.shape
    m, n = pos_bias.shape
    w = memory.transpose(1, 0, 2).reshape(n * m, d)
    pb_t = pos_bias.T
    grid = (l // _TILE,)
    out = pl.pallas_call(
        _fused,
        grid=grid,
        in_specs=[
            pl.BlockSpec((_TILE, d), lambda i: (i, 0)),
            pl.BlockSpec((n * m, d), lambda i: (0, 0)),
            pl.BlockSpec((n, m), lambda i: (0, 0)),
        ],
        out_specs=pl.BlockSpec((_TILE, d), lambda i: (i, 0)),
        scratch_shapes=[pltpu.VMEM((2, 2 * m), jnp.float32)],
        out_shape=jax.ShapeDtypeStruct((l, d), jnp.float32),
    )(x[0], w, pb_t)
    return out[None]


# speedup vs baseline: 1.2403x; 1.0291x over previous
import jax
import jax.numpy as jnp
from jax.experimental import pallas as pl
from jax.experimental.pallas import tpu as pltpu

_TILE = 1024


def _fused(x_ref, w_ref, pb_ref, out_ref, carry_ref):
    i = pl.program_id(0)
    t = x_ref.shape[0]
    m = pb_ref.shape[1]

    @pl.when(i == 0)
    def _():
        carry_ref[...] = jnp.zeros_like(carry_ref)

    w = w_ref[...]
    y = jax.lax.dot_general(x_ref[...], w, (((1,), (1,)), ((), ())),
                            preferred_element_type=jnp.float32)
    pb = pb_ref[...]
    prev = carry_ref[...]
    full01 = jnp.concatenate([prev, y[:, 0:2 * m]], axis=0)
    carry_ref[...] = y[t - 2:t, 0:2 * m]

    s0 = full01[0:t, 0:m] + pb[0][None, :]
    s1 = full01[1:t + 1, m:2 * m] + pb[1][None, :]
    s2 = y[:, 2 * m:3 * m] + pb[2][None, :]

    o0 = (s0 >= s1) & (s0 >= s2)
    o1 = jnp.logical_not(o0) & (s1 >= s2)
    f = jnp.concatenate([o0.astype(jnp.bfloat16),
                         o1.astype(jnp.bfloat16)], axis=1)

    mem2 = w[2 * m:3 * m]
    dcat = (w[0:2 * m] - jnp.concatenate([mem2, mem2], axis=0)
            ).astype(jnp.bfloat16)
    base = jnp.sum(mem2, axis=0)[None, :]

    out = jax.lax.dot_general(f, dcat, (((1,), (0,)), ((), ())),
                              preferred_element_type=jnp.float32)
    out_ref[...] = out + base


def kernel(x, memory, pos_bias):
    b, l, d = x.shape
    m, n = pos_bias.shape
    w = memory.transpose(1, 0, 2).reshape(n * m, d)
    pb_t = pos_bias.T
    grid = (l // _TILE,)
    out = pl.pallas_call(
        _fused,
        grid=grid,
        in_specs=[
            pl.BlockSpec((_TILE, d), lambda i: (i, 0)),
            pl.BlockSpec((n * m, d), lambda i: (0, 0)),
            pl.BlockSpec((n, m), lambda i: (0, 0)),
        ],
        out_specs=pl.BlockSpec((_TILE, d), lambda i: (i, 0)),
        scratch_shapes=[pltpu.VMEM((2, 2 * m), jnp.float32)],
        out_shape=jax.ShapeDtypeStruct((l, d), jnp.float32),
    )(x[0], w, pb_t)
    return out[None]


# speedup vs baseline: 1.2418x; 1.0013x over previous
import jax
import jax.numpy as jnp
from jax.experimental import pallas as pl
from jax.experimental.pallas import tpu as pltpu

_TILE = 1024


def _fused(x_ref, xb_ref, w_ref, pb_ref, out_ref):
    i = pl.program_id(0)
    t = x_ref.shape[0]
    m = pb_ref.shape[1]

    w = w_ref[...]
    y = jax.lax.dot_general(x_ref[...], w, (((1,), (1,)), ((), ())),
                            preferred_element_type=jnp.float32)
    pb = pb_ref[...]

    yb = jax.lax.dot_general(xb_ref[...], w, (((1,), (1,)), ((), ())),
                             preferred_element_type=jnp.float32)
    prev = jnp.where(i == 0, 0.0, yb[6:8, 0:2 * m])
    full01 = jnp.concatenate([prev, y[:, 0:2 * m]], axis=0)

    s0 = full01[0:t, 0:m] + pb[0][None, :]
    s1 = full01[1:t + 1, m:2 * m] + pb[1][None, :]
    s2 = y[:, 2 * m:3 * m] + pb[2][None, :]

    o0 = (s0 >= s1) & (s0 >= s2)
    o1 = jnp.logical_not(o0) & (s1 >= s2)
    f = jnp.concatenate([o0.astype(jnp.bfloat16),
                         o1.astype(jnp.bfloat16)], axis=1)

    mem2 = w[2 * m:3 * m]
    dcat = (w[0:2 * m] - jnp.concatenate([mem2, mem2], axis=0)
            ).astype(jnp.bfloat16)
    base = jnp.sum(mem2, axis=0)[None, :]

    out = jax.lax.dot_general(f, dcat, (((1,), (0,)), ((), ())),
                              preferred_element_type=jnp.float32)
    out_ref[...] = out + base


def kernel(x, memory, pos_bias):
    b, l, d = x.shape
    m, n = pos_bias.shape
    w = memory.transpose(1, 0, 2).reshape(n * m, d)
    pb_t = pos_bias.T
    nb = _TILE // 8
    out = pl.pallas_call(
        _fused,
        grid=(l // _TILE,),
        in_specs=[
            pl.BlockSpec((_TILE, d), lambda i: (i, 0)),
            pl.BlockSpec((8, d), lambda i: (jnp.maximum(i * nb - 1, 0), 0)),
            pl.BlockSpec((n * m, d), lambda i: (0, 0)),
            pl.BlockSpec((n, m), lambda i: (0, 0)),
        ],
        out_specs=pl.BlockSpec((_TILE, d), lambda i: (i, 0)),
        compiler_params=pltpu.CompilerParams(
            dimension_semantics=("parallel",)),
        out_shape=jax.ShapeDtypeStruct((l, d), jnp.float32),
    )(x[0], x[0], w, pb_t)
    return out[None]
